# baseline (speedup 1.0000x reference)
import jax
import jax.numpy as jnp
from jax.experimental import pallas as pl
from jax.experimental.pallas import tpu as pltpu

B, L, D = 16, 4096, 1024
NL = 8
LBLK = L // NL
NTILE = D // 128


def _body(mask_ref, feats_ref, out_ref, amax_ref, asum_ref, acnt_ref):
    l = pl.program_id(1)
    mf = mask_ref[0].astype(jnp.float32)
    mb = mf > 0.5
    neg = jnp.float32(-1e30)
    bcnt = jnp.sum(mf) * jnp.float32(1.0 / 128.0)

    @pl.when(l == 0)
    def _():
        acnt_ref[0] = bcnt
        for j in range(NTILE):
            xj = feats_ref[0, :, j * 128:(j + 1) * 128]
            amax_ref[0, j * 128:(j + 1) * 128] = jnp.max(
                jnp.where(mb, xj, neg), axis=0)
            asum_ref[0, j * 128:(j + 1) * 128] = jnp.sum(
                jnp.where(mb, xj, 0.0), axis=0)

    @pl.when(l > 0)
    def _():
        acnt_ref[0] = acnt_ref[0] + bcnt
        for j in range(NTILE):
            xj = feats_ref[0, :, j * 128:(j + 1) * 128]
            amax_ref[0, j * 128:(j + 1) * 128] = jnp.maximum(
                amax_ref[0, j * 128:(j + 1) * 128],
                jnp.max(jnp.where(mb, xj, neg), axis=0))
            asum_ref[0, j * 128:(j + 1) * 128] = (
                asum_ref[0, j * 128:(j + 1) * 128]
                + jnp.sum(jnp.where(mb, xj, 0.0), axis=0))

    @pl.when(l == NL - 1)
    def _():
        out_ref[0, 0, :D] = amax_ref[0]
        out_ref[0, 0, D:] = asum_ref[0] / acnt_ref[0]


def kernel(feats, mask):
    mask128 = jnp.broadcast_to(
        mask[:, :, None], (B, L, 128)).astype(jnp.int8)
    out = pl.pallas_call(
        _body,
        grid=(B, NL),
        in_specs=[
            pl.BlockSpec((1, LBLK, 128), lambda b, l: (b, l, 0)),
            pl.BlockSpec((1, LBLK, D), lambda b, l: (b, l, 0)),
        ],
        out_specs=pl.BlockSpec((1, 1, 2 * D), lambda b, l: (b, 0, 0)),
        out_shape=jax.ShapeDtypeStruct((B, 1, 2 * D), jnp.float32),
        scratch_shapes=[
            pltpu.VMEM((1, D), jnp.float32),
            pltpu.VMEM((1, D), jnp.float32),
            pltpu.SMEM((1,), jnp.float32),
        ],
    )(mask128, feats)
    return out.reshape(B, 2 * D)
